# Initial kernel scaffold; baseline (speedup 1.0000x reference)
import jax
import jax.numpy as jnp
from jax.experimental import pallas as pl


def kernel(xs, table):
    raise NotImplementedError("write your pallas kernel here")


# speedup vs baseline: 2.8039x; 2.8039x over previous
import functools

import jax
import jax.numpy as jnp
from jax import lax
from jax.experimental import pallas as pl
from jax.experimental.pallas import tpu as pltpu
from jax.experimental.pallas import tpu_sc as plsc

_B, _H, _D, _V = 16384, 50, 64, 1000000
_NC, _NS, _L = 2, 16, 16
_NW = _NC * _NS
_BPW = _B // _NW
_C = 8
_CW = _C * _H
_NCH = _BPW // _C
_INV = 1.0 / _H


def _body(xs_hbm, table_hbm, out_hbm, idx_v, rows0, rows1, out_v, sem0, sem1):
    wid = lax.axis_index("s") * _NC + lax.axis_index("c")
    base = wid * _BPW

    pltpu.sync_copy(xs_hbm.at[pl.ds(base * _H, _BPW * _H)], idx_v)
    pltpu.async_copy(table_hbm.at[idx_v.at[pl.ds(0, _CW)]], rows0, sem0)
    pltpu.async_copy(table_hbm.at[idx_v.at[pl.ds(_CW, _CW)]], rows1, sem1)

    @pl.loop(0, _NCH, step=2)
    def _chunks(ci):
        for b in range(2):
            rows = rows0 if b == 0 else rows1
            sem = sem0 if b == 0 else sem1
            cur = ci + b
            pltpu.make_async_copy(
                table_hbm.at[idx_v.at[pl.ds(cur * _CW, _CW)]], rows, sem
            ).wait()

            @pl.loop(0, _C)
            def _items(i, rows=rows, cur=cur):
                out_row = cur * _C + i
                rowbase = i * _H
                z = jnp.zeros((_L,), jnp.float32)

                @pl.loop(0, _H, init_carry=(z, z, z, z), unroll=2)
                def _acc(j, carry, rows=rows, rowbase=rowbase):
                    a0, a1, a2, a3 = carry
                    rr = rowbase + j
                    return (
                        a0 + rows[rr, pl.ds(0, _L)],
                        a1 + rows[rr, pl.ds(_L, _L)],
                        a2 + rows[rr, pl.ds(2 * _L, _L)],
                        a3 + rows[rr, pl.ds(3 * _L, _L)],
                    )

                a0, a1, a2, a3 = _acc
                out_v[out_row, pl.ds(0, _L)] = a0 * _INV
                out_v[out_row, pl.ds(_L, _L)] = a1 * _INV
                out_v[out_row, pl.ds(2 * _L, _L)] = a2 * _INV
                out_v[out_row, pl.ds(3 * _L, _L)] = a3 * _INV

            nxt = cur + 2

            @pl.when(nxt < _NCH)
            def _fire(rows=rows, sem=sem, nxt=nxt):
                pltpu.async_copy(
                    table_hbm.at[idx_v.at[pl.ds(nxt * _CW, _CW)]], rows, sem
                )

    pltpu.sync_copy(out_v, out_hbm.at[pl.ds(base, _BPW)])


@functools.cache
def _make_kernel():
    mesh = plsc.VectorSubcoreMesh(
        core_axis_name="c", subcore_axis_name="s",
        num_cores=_NC, num_subcores=_NS,
    )
    return pl.kernel(
        _body,
        out_type=jax.ShapeDtypeStruct((_B, _D), jnp.float32),
        mesh=mesh,
        scratch_types=[
            pltpu.VMEM((_BPW * _H,), jnp.int32),
            pltpu.VMEM((_CW, _D), jnp.float32),
            pltpu.VMEM((_CW, _D), jnp.float32),
            pltpu.VMEM((_BPW, _D), jnp.float32),
            pltpu.SemaphoreType.DMA,
            pltpu.SemaphoreType.DMA,
        ],
        compiler_params=pltpu.CompilerParams(use_tc_tiling_on_sc=False),
    )


def kernel(xs, table):
    xs_flat = xs.reshape(-1).astype(jnp.int32)
    return _make_kernel()(xs_flat, table)
